# baseline (speedup 1.0000x reference)
import functools

import jax
import jax.numpy as jnp
from jax import lax
from jax.experimental import pallas as pl
from jax.experimental.pallas import tpu as pltpu
from jax.experimental.pallas import tpu_sc as plsc

BATCH = 16384
DIM = 64
NUM_CORES = 2
NUM_SUBCORES = 16
NUM_WORKERS = NUM_CORES * NUM_SUBCORES
BPW = BATCH // NUM_WORKERS
LANES = 16
CH = 32
NCH = BPW // CH


def _sqrt16(x):
    i = lax.bitcast_convert_type(x, jnp.int32)
    i = jnp.int32(0x5F3759DF) - lax.shift_right_arithmetic(i, jnp.int32(1))
    r = lax.bitcast_convert_type(i, jnp.float32)
    half = x * jnp.float32(0.5)
    for _ in range(4):
        r = r * (jnp.float32(1.5) - half * r * r)
    return x * r


def _transe_body(ent_hbm, rel_hbm, h_hbm, r_hbm, t_hbm, out_hbm,
                 hfull, rfull, tfull, hbuf, rbuf, tbuf, out_v, sem):
    wid = lax.axis_index("s") * NUM_CORES + lax.axis_index("c")
    base = wid * BPW

    pltpu.sync_copy(h_hbm.at[pl.ds(base, BPW)], hfull)
    pltpu.sync_copy(r_hbm.at[pl.ds(base, BPW)], rfull)
    pltpu.sync_copy(t_hbm.at[pl.ds(base, BPW)], tfull)

    lanes = lax.iota(jnp.int32, LANES)
    perms = [lanes ^ sh for sh in (8, 4, 2, 1)]

    def chunk_body(k, carry):
        off = k * CH
        hidx_lo = hfull[pl.ds(off, LANES)]
        hidx_hi = hfull[pl.ds(off + LANES, LANES)]
        ridx_lo = rfull[pl.ds(off, LANES)]
        ridx_hi = rfull[pl.ds(off + LANES, LANES)]
        tidx_lo = tfull[pl.ds(off, LANES)]
        tidx_hi = tfull[pl.ds(off + LANES, LANES)]
        for j in range(CH):
            hi = hidx_lo[j] if j < LANES else hidx_hi[j - LANES]
            ri = ridx_lo[j] if j < LANES else ridx_hi[j - LANES]
            ti = tidx_lo[j] if j < LANES else tidx_hi[j - LANES]
            pltpu.async_copy(ent_hbm.at[hi], hbuf.at[j], sem)
            pltpu.async_copy(rel_hbm.at[ri], rbuf.at[j], sem)
            pltpu.async_copy(ent_hbm.at[ti], tbuf.at[j], sem)
        pltpu.make_async_copy(ent_hbm.at[pl.ds(0, CH)], hbuf, sem).wait()
        pltpu.make_async_copy(rel_hbm.at[pl.ds(0, CH)], rbuf, sem).wait()
        pltpu.make_async_copy(ent_hbm.at[pl.ds(0, CH)], tbuf, sem).wait()

        for g in range(CH // LANES):
            vec = jnp.zeros((LANES,), jnp.float32)
            for jj in range(LANES):
                j = g * LANES + jj
                acc = jnp.zeros((LANES,), jnp.float32)
                for c in range(DIM // LANES):
                    hv = hbuf[j, pl.ds(c * LANES, LANES)]
                    rv = rbuf[j, pl.ds(c * LANES, LANES)]
                    tv = tbuf[j, pl.ds(c * LANES, LANES)]
                    d = (hv - tv) + rv
                    acc = acc + d * d
                for p in perms:
                    acc = acc + acc.at[p].get(mode="promise_in_bounds")
                vec = jnp.where(lanes == jj, acc, vec)
            out_v[pl.ds(off + g * LANES, LANES)] = _sqrt16(vec)
        return carry

    lax.fori_loop(0, NCH, chunk_body, jnp.int32(0))

    pltpu.sync_copy(out_v, out_hbm.at[pl.ds(base, BPW)])


@jax.jit
def kernel(entity_emb, relation_emb, h, r, t):
    mesh = plsc.VectorSubcoreMesh(core_axis_name="c", subcore_axis_name="s")
    f = functools.partial(
        pl.kernel,
        mesh=mesh,
        out_type=jax.ShapeDtypeStruct((BATCH,), jnp.float32),
        scratch_types=[
            pltpu.VMEM((BPW,), jnp.int32),
            pltpu.VMEM((BPW,), jnp.int32),
            pltpu.VMEM((BPW,), jnp.int32),
            pltpu.VMEM((CH, DIM), jnp.float32),
            pltpu.VMEM((CH, DIM), jnp.float32),
            pltpu.VMEM((CH, DIM), jnp.float32),
            pltpu.VMEM((BPW,), jnp.float32),
            pltpu.SemaphoreType.DMA,
        ],
    )(_transe_body)
    return f(entity_emb, relation_emb, h, r.astype(jnp.int32), t)
